# baseline (speedup 1.0000x reference)
import functools

import jax
import jax.numpy as jnp
from jax import lax
from jax.experimental import pallas as pl
from jax.experimental.pallas import tpu as pltpu
from jax.experimental.pallas import tpu_sc as plsc

N = 10000
E = 320000
EH = E // 2
D = 128
H = 8
DH = 16
CLAMP = 5.0

G = 40
NTILES = 32
RPT = EH // NTILES
NCHUNK = RPT // G
ROWS_PT = 624
TAIL0 = 16 * ROWS_PT
TAILR = N - TAIL0

_f32 = jnp.float32


def _sc_body(src_hbm, dst_hbm, kvtab, qtab, epr, awt,
             we_out, pvpr_out, den_out,
             srcb0A, srcb1A, dstb0A, dstb1A,
             srcb0B, srcb1B, dstb0B, dstb1B,
             kvbuf0, kvbuf1, qbuf0, qbuf1, ebufA, ebufB, owbuf,
             pprb0, pprb1, denb0, denb1, awb,
             pvpr_sh, den_sh,
             sem_i, sem_g0, sem_g1, sem_e, sem_s0, sem_s1, sem_w):
    c = lax.axis_index("c")
    s = lax.axis_index("s")
    wid = c * 16 + s
    lanes = lax.iota(jnp.int32, 16)
    zero16 = jnp.zeros((16,), _f32)

    pltpu.sync_copy(awt, awb)

    def _zrow(i, _):
        for t in range(8):
            pprb0[i, pl.ds(t * 16, 16)] = zero16
            pprb1[i, pl.ds(t * 16, 16)] = zero16
        denb0[i, :] = zero16
        denb1[i, :] = zero16
        return 0
    lax.fori_loop(0, G, _zrow, 0)

    row0 = s * ROWS_PT
    for t in range(15):
        pltpu.sync_copy(pprb0, pvpr_sh.at[pl.ds(row0 + t * G, G)])
        pltpu.sync_copy(denb0, den_sh.at[pl.ds(row0 + t * G, G)])
    pltpu.sync_copy(pprb0, pvpr_sh.at[pl.ds(row0 + ROWS_PT - G, G)])
    pltpu.sync_copy(denb0, den_sh.at[pl.ds(row0 + ROWS_PT - G, G)])

    @pl.when(s == 0)
    def _zero_tail():
        pltpu.sync_copy(pprb0.at[pl.ds(0, TAILR)],
                        pvpr_sh.at[pl.ds(TAIL0, TAILR)])
        pltpu.sync_copy(denb0.at[pl.ds(0, TAILR)],
                        den_sh.at[pl.ds(TAIL0, TAILR)])

    plsc.subcore_barrier()

    rbase0 = wid * RPT
    _gdn = lax.GatherDimensionNumbers(
        offset_dims=(), collapsed_slice_dims=(0,), start_index_map=(0,))
    perms = [(lanes ^ sft)[:, None] for sft in (8, 4, 2, 1)]

    def _lanesum(t):
        for idxp in perms:
            t = t + lax.gather(t, idxp, _gdn, slice_sizes=(1,),
                               mode=lax.GatherScatterMode.PROMISE_IN_BOUNDS)
        return t

    himask = jnp.full((16,), -65536, jnp.int32)
    aws = [awb[j, :] for j in range(4)]
    hmasks = [jnp.where(lanes == j, 1.0, 0.0).astype(_f32) for j in range(4)]

    def _compute(kvbuf, qbuf, pprb, denb, ebuf, g):
        @plsc.parallel_loop(0, G, unroll=4)
        def edge_body(i):
            dvec = zero16
            for j in range(4):
                kv = kvbuf[i, pl.ds(j * 16, 16)]
                k = jax.lax.bitcast_convert_type(kv << 16, _f32)
                v = jax.lax.bitcast_convert_type(kv & himask, _f32)
                q = qbuf[i, pl.ds(j * 16, 16)]
                ee = ebuf[i, pl.ds(g * 64 + j * 16, 16)]
                ew_w = jax.lax.bitcast_convert_type(ee << 16, _f32)
                ew_b = jax.lax.bitcast_convert_type(ee & himask, _f32)
                ew = jnp.maximum((k + q) * ew_w + ew_b, 0.0)
                owbuf[i, pl.ds(g * 64 + j * 16, 16)] = ew
                sv = _lanesum(ew * aws[j])
                sv = jnp.minimum(jnp.maximum(sv, -CLAMP), CLAMP)
                pvec = jnp.exp(sv)
                pprb[i, pl.ds(j * 16, 16)] = v * pvec
                pprb[i, pl.ds(64 + j * 16, 16)] = ew * pvec
                dvec = dvec + pvec * hmasks[j]
            denb[i, :] = dvec

    IDX = {0: (srcb0A, srcb1A, dstb0A, dstb1A),
           1: (srcb0B, srcb1B, dstb0B, dstb1B)}
    EB = {0: ebufA, 1: ebufB}

    def fire_idx(tk, p):
        s0b, s1b, d0b, d1b = IDX[p]
        rb = rbase0 + tk * G
        pltpu.async_copy(src_hbm.at[pl.ds(rb, G)], s0b, sem_i)
        pltpu.async_copy(src_hbm.at[pl.ds(rb + EH, G)], s1b, sem_i)
        pltpu.async_copy(dst_hbm.at[pl.ds(rb, G)], d0b, sem_i)
        pltpu.async_copy(dst_hbm.at[pl.ds(rb + EH, G)], d1b, sem_i)

    def wait_idx(p):
        for b in IDX[p]:
            pltpu.make_async_copy(src_hbm.at[pl.ds(0, G)], b, sem_i).wait()

    def fire_epr(tk, p):
        pltpu.async_copy(epr.at[pl.ds(rbase0 + tk * G, G)], EB[p], sem_e)

    def wait_epr(p):
        pltpu.make_async_copy(epr.at[pl.ds(0, G)], EB[p], sem_e).wait()

    def fire_g(p):
        s0b, s1b, d0b, d1b = IDX[p]
        pltpu.async_copy(kvtab.at[s0b], kvbuf0, sem_g0)
        pltpu.async_copy(qtab.at[d0b], qbuf0, sem_g0)
        pltpu.async_copy(kvtab.at[s1b], kvbuf1, sem_g1)
        pltpu.async_copy(qtab.at[d1b], qbuf1, sem_g1)

    def wait_g(g):
        kvb, qb, sem = (kvbuf0, qbuf0, sem_g0) if g == 0 else \
                       (kvbuf1, qbuf1, sem_g1)
        pltpu.make_async_copy(kvtab.at[srcb0A], kvb, sem).wait()
        pltpu.make_async_copy(qtab.at[dstb0A], qb, sem).wait()

    def fire_s(g, p):
        d0b, d1b = IDX[p][2], IDX[p][3]
        db = d0b if g == 0 else d1b
        pprb, denb, sem = (pprb0, denb0, sem_s0) if g == 0 else \
                          (pprb1, denb1, sem_s1)
        pltpu.async_copy(pprb, pvpr_sh.at[db], sem, add=True)
        pltpu.async_copy(denb, den_sh.at[db], sem, add=True)

    def wait_s(g):
        pprb, denb, sem = (pprb0, denb0, sem_s0) if g == 0 else \
                          (pprb1, denb1, sem_s1)
        pltpu.make_async_copy(pprb, pvpr_sh.at[dstb0A], sem).wait()
        pltpu.make_async_copy(denb, den_sh.at[dstb0A], sem).wait()

    def fire_w(tk):
        pltpu.async_copy(owbuf, we_out.at[pl.ds(rbase0 + tk * G, G)], sem_w)

    def wait_w():
        pltpu.make_async_copy(owbuf, we_out.at[pl.ds(0, G)], sem_w).wait()

    def do_chunk(tk, cur, nxt, last):
        if not last:
            fire_idx(tk + 1, nxt)
        wait_w()
        wait_g(0)
        wait_epr(cur)
        wait_s(0)
        _compute(kvbuf0, qbuf0, pprb0, denb0, EB[cur], 0)
        fire_s(0, cur)
        if not last:
            wait_idx(nxt)
            fire_epr(tk + 1, nxt)
        wait_g(1)
        wait_s(1)
        _compute(kvbuf1, qbuf1, pprb1, denb1, EB[cur], 1)
        fire_s(1, cur)
        fire_w(tk)
        if not last:
            fire_g(nxt)

    fire_idx(0, 0)
    wait_idx(0)
    fire_s(0, 0)
    fire_s(1, 0)
    fire_w(0)
    fire_epr(0, 0)
    fire_g(0)

    def pair_body(m, _):
        do_chunk(2 * m, 0, 1, False)
        do_chunk(2 * m + 1, 1, 0, False)
        return 0

    lax.fori_loop(0, (NCHUNK - 1) // 2, pair_body, 0)
    do_chunk(NCHUNK - 1, 0, 1, True)

    wait_s(0)
    wait_s(1)
    wait_w()
    plsc.subcore_barrier()

    out_r0 = c * N + s * ROWS_PT
    pltpu.sync_copy(pvpr_sh.at[pl.ds(row0, ROWS_PT)],
                    pvpr_out.at[pl.ds(out_r0, ROWS_PT)])
    pltpu.sync_copy(den_sh.at[pl.ds(row0, ROWS_PT)],
                    den_out.at[pl.ds(out_r0, ROWS_PT)])

    @pl.when(s == 0)
    def _flush_tail():
        tr0 = c * N + TAIL0
        pltpu.sync_copy(pvpr_sh.at[pl.ds(TAIL0, TAILR)],
                        pvpr_out.at[pl.ds(tr0, TAILR)])
        pltpu.sync_copy(den_sh.at[pl.ds(TAIL0, TAILR)],
                        den_out.at[pl.ds(tr0, TAILR)])


_sc_edge = pl.kernel(
    _sc_body,
    out_type=[
        jax.ShapeDtypeStruct((EH, 128), _f32),
        jax.ShapeDtypeStruct((2 * N, 128), _f32),
        jax.ShapeDtypeStruct((2 * N, 16), _f32),
    ],
    mesh=plsc.VectorSubcoreMesh(core_axis_name="c", subcore_axis_name="s"),
    compiler_params=pltpu.CompilerParams(use_tc_tiling_on_sc=False),
    scratch_types=[
        pltpu.VMEM((G,), jnp.int32),
        pltpu.VMEM((G,), jnp.int32),
        pltpu.VMEM((G,), jnp.int32),
        pltpu.VMEM((G,), jnp.int32),
        pltpu.VMEM((G,), jnp.int32),
        pltpu.VMEM((G,), jnp.int32),
        pltpu.VMEM((G,), jnp.int32),
        pltpu.VMEM((G,), jnp.int32),
        pltpu.VMEM((G, 64), jnp.int32),
        pltpu.VMEM((G, 64), jnp.int32),
        pltpu.VMEM((G, 64), _f32),
        pltpu.VMEM((G, 64), _f32),
        pltpu.VMEM((G, 128), jnp.int32),
        pltpu.VMEM((G, 128), jnp.int32),
        pltpu.VMEM((G, 128), _f32),
        pltpu.VMEM((G, 128), _f32),
        pltpu.VMEM((G, 128), _f32),
        pltpu.VMEM((G, 16), _f32),
        pltpu.VMEM((G, 16), _f32),
        pltpu.VMEM((4, 16), _f32),
        pltpu.VMEM_SHARED((N, 128), _f32),
        pltpu.VMEM_SHARED((N, 16), _f32),
        pltpu.SemaphoreType.DMA,
        pltpu.SemaphoreType.DMA,
        pltpu.SemaphoreType.DMA,
        pltpu.SemaphoreType.DMA,
        pltpu.SemaphoreType.DMA,
        pltpu.SemaphoreType.DMA,
        pltpu.SemaphoreType.DMA,
    ],
)


_BN_ = 1000
_BE_ = 1000


def _pack_bf16(lo, hi):
    lb = jax.lax.bitcast_convert_type(lo, jnp.uint32)
    hb = jax.lax.bitcast_convert_type(hi, jnp.uint32)
    lr = (lb + jnp.uint32(0x7FFF) + ((lb >> 16) & jnp.uint32(1))) >> 16
    hr = (hb + jnp.uint32(0x7FFF) + ((hb >> 16) & jnp.uint32(1))) >> 16
    return jax.lax.bitcast_convert_type((hr << 16) | lr, jnp.int32)


def _proj_body(x, qw, kw, vw, qb, q0, q1, kv0, kv1):
    xb = x[...]
    qh = jnp.dot(xb, qw[...], preferred_element_type=_f32) + qb[...]
    kh = jnp.dot(xb, kw[...], preferred_element_type=_f32)
    vh = jnp.dot(xb, vw[...], preferred_element_type=_f32)
    q0[...] = qh[:, :64]
    q1[...] = qh[:, 64:]
    kvp = _pack_bf16(kh, vh)
    kv0[...] = kvp[:, :64]
    kv1[...] = kvp[:, 64:]


def _proj(x, Qw, Kw, Vw, Qb):
    nb = N // _BN_
    blk = lambda w: pl.BlockSpec((_BN_, w), lambda i: (i, 0))
    full = lambda a, b: pl.BlockSpec((a, b), lambda i: (0, 0))
    return pl.pallas_call(
        _proj_body,
        grid=(nb,),
        in_specs=[blk(128), full(128, 128), full(128, 128), full(128, 128),
                  full(1, 128)],
        out_specs=[blk(64)] * 4,
        out_shape=[jax.ShapeDtypeStruct((N, 64), _f32)] * 2
        + [jax.ShapeDtypeStruct((N, 64), jnp.int32)] * 2,
    )(x, Qw, Kw, Vw, Qb)


def _epr_body(ea_lo, ea_hi, ew, eb, o0, o1):
    m_lo = jnp.dot(ea_lo[...], ew[...], preferred_element_type=_f32) + eb[...]
    m_hi = jnp.dot(ea_hi[...], ew[...], preferred_element_type=_f32) + eb[...]
    o0[...] = jnp.concatenate(
        [_pack_bf16(m_lo[:, 0:64], m_lo[:, 64:128]),
         _pack_bf16(m_hi[:, 0:64], m_hi[:, 64:128])], axis=1)
    o1[...] = jnp.concatenate(
        [_pack_bf16(m_lo[:, 128:192], m_lo[:, 192:256]),
         _pack_bf16(m_hi[:, 128:192], m_hi[:, 192:256])], axis=1)


def _epr(edge_attr, Ewp, Ebp):
    nb = EH // _BE_
    return pl.pallas_call(
        _epr_body,
        grid=(nb,),
        in_specs=[pl.BlockSpec((_BE_, 128), lambda i: (i, 0)),
                  pl.BlockSpec((_BE_, 128), lambda i: (i + EH // _BE_, 0)),
                  pl.BlockSpec((128, 256), lambda i: (0, 0)),
                  pl.BlockSpec((1, 256), lambda i: (0, 0))],
        out_specs=[pl.BlockSpec((_BE_, 128), lambda i: (i, 0))] * 2,
        out_shape=[jax.ShapeDtypeStruct((EH, 128), jnp.int32)] * 2,
    )(edge_attr, edge_attr, Ewp, Ebp)


def _epass1_body(ea_lo, ea_hi, w0, w1, m0, m1, ob, eprep, stats):
    i = pl.program_id(0)
    w0b = w0[...]
    w1b = w1[...]
    m_lo = (ea_lo[...]
            + jnp.dot(w0b[:, :64], m0[...], preferred_element_type=_f32)
            + jnp.dot(w1b[:, :64], m1[...], preferred_element_type=_f32)
            + ob[...])
    m_hi = (ea_hi[...]
            + jnp.dot(w0b[:, 64:], m0[...], preferred_element_type=_f32)
            + jnp.dot(w1b[:, 64:], m1[...], preferred_element_type=_f32)
            + ob[...])
    eprep[...] = jnp.concatenate([m_lo, m_hi], axis=1).astype(jnp.bfloat16)

    @pl.when(i == 0)
    def _():
        stats[...] = jnp.zeros((8, 128), _f32)

    s0 = (jnp.sum(m_lo, axis=0, keepdims=True)
          + jnp.sum(m_hi, axis=0, keepdims=True))
    s1 = (jnp.sum(m_lo * m_lo, axis=0, keepdims=True)
          + jnp.sum(m_hi * m_hi, axis=0, keepdims=True))
    stats[0:1, :] += s0
    stats[1:2, :] += s1


def _epass1(edge_attr, we0, we1, W03, W47, oeb):
    nbh = EH // _BE_
    return pl.pallas_call(
        _epass1_body,
        grid=(nbh,),
        in_specs=[pl.BlockSpec((_BE_, 128), lambda i: (i, 0)),
                  pl.BlockSpec((_BE_, 128), lambda i: (i + EH // _BE_, 0)),
                  pl.BlockSpec((_BE_, 128), lambda i: (i, 0)),
                  pl.BlockSpec((_BE_, 128), lambda i: (i, 0)),
                  pl.BlockSpec((64, 128), lambda i: (0, 0)),
                  pl.BlockSpec((64, 128), lambda i: (0, 0)),
                  pl.BlockSpec((1, 128), lambda i: (0, 0))],
        out_specs=[pl.BlockSpec((_BE_, 256), lambda i: (i, 0)),
                   pl.BlockSpec((8, 128), lambda i: (0, 0))],
        out_shape=[jax.ShapeDtypeStruct((EH, 256), jnp.bfloat16),
                   jax.ShapeDtypeStruct((8, 128), _f32)],
    )(edge_attr, edge_attr, we0, we1, W03, W47, oeb)


def _epass2_body(epre, sc, sh, e):
    e[...] = epre[...].astype(_f32) * sc[...] + sh[...]


def _epass2(epre, scale, shift):
    nb = E // _BE_
    nbh = EH // _BE_
    return pl.pallas_call(
        _epass2_body,
        grid=(nb,),
        in_specs=[pl.BlockSpec((_BE_, 128), lambda i: (i % nbh, i // nbh)),
                  pl.BlockSpec((1, 128), lambda i: (0, 0)),
                  pl.BlockSpec((1, 128), lambda i: (0, 0))],
        out_specs=pl.BlockSpec((_BE_, 128), lambda i: (i, 0)),
        out_shape=jax.ShapeDtypeStruct((E, 128), _f32),
    )(epre, scale, shift)


def _node_body(x, pv, pr, invb, degc, vebd, ohw, ohb, dc0, dc1,
               f1w, f1b, f2w, f2b, g1, b1, g2, b2, out):
    xb = x[...]
    wv = pv[...] * invb[...] + jnp.dot(pr[...] * invb[...], vebd[...],
                                       preferred_element_type=_f32)
    ld = jnp.log(degc[...] + 1.0)
    hh = wv * dc0[...] + (wv * ld) * dc1[...]
    hh = jnp.dot(hh, ohw[...], preferred_element_type=_f32) + ohb[...]
    t = xb + hh
    mu = jnp.mean(t, axis=0, keepdims=True)
    var = jnp.mean(t * t, axis=0, keepdims=True) - mu * mu
    hn = g1[...] * (t - mu) / jnp.sqrt(var + 1e-5) + b1[...]
    u = jnp.maximum(jnp.dot(hn, f1w[...], preferred_element_type=_f32)
                    + f1b[...], 0.0)
    u = jnp.dot(u, f2w[...], preferred_element_type=_f32) + f2b[...] + hn
    mu2 = jnp.mean(u, axis=0, keepdims=True)
    var2 = jnp.mean(u * u, axis=0, keepdims=True) - mu2 * mu2
    out[...] = g2[...] * (u - mu2) / jnp.sqrt(var2 + 1e-5) + b2[...]


def _node(x, pv, pr, invb, degc, vebd, ohw, ohb, dc0, dc1,
          f1w, f1b, f2w, f2b, g1, b1, g2, b2):
    return pl.pallas_call(
        _node_body,
        out_shape=jax.ShapeDtypeStruct((N, 128), _f32),
    )(x, pv, pr, invb, degc, vebd, ohw, ohb, dc0, dc1,
      f1w, f1b, f2w, f2b, g1, b1, g2, b2)


def kernel(x, edge_attr, edge_index, deg, Qw, Qb, Kw, Ew, Eb, Vw, Aw, VeRow,
           out_h_w, out_h_b, out_e_w, out_e_b, deg_coef, g1h, b1h, g1e, b1e,
           fc1w, fc1b, fc2w, fc2b, g2h, b2h):
    src = edge_index[0]
    dst = edge_index[1]

    q0, q1, kv0, kv1 = _proj(x, Qw, Kw, Vw, Qb.reshape(1, 128))

    perm = jnp.array(
        [32 * h + t for h in range(4) for t in range(16)]
        + [32 * h + 16 + t for h in range(4) for t in range(16)]
        + [32 * h + t for h in range(4, 8) for t in range(16)]
        + [32 * h + 16 + t for h in range(4, 8) for t in range(16)],
        dtype=jnp.int32)
    Ewp = Ew[:, perm]
    Ebp = Eb[perm].reshape(1, 256)
    eprp0, eprp1 = _epr(edge_attr, Ewp, Ebp)

    awt = jnp.transpose(Aw[:, :, 0], (1, 0))

    we0, pvpr0, den0 = _sc_edge(src, dst, kv0, q0, eprp0, awt[0:4])
    we1, pvpr1, den1 = _sc_edge(src, dst, kv1, q1, eprp1, awt[4:8])

    a0 = pvpr0[:N] + pvpr0[N:]
    a1 = pvpr1[:N] + pvpr1[N:]
    pv = jnp.concatenate([a0[:, :64], a1[:, :64]], axis=1)
    pr = jnp.concatenate([a0[:, 64:], a1[:, 64:]], axis=1)
    den = jnp.concatenate([(den0[:N] + den0[N:])[:, :4],
                           (den1[:N] + den1[N:])[:, :4]], axis=1)
    inv = jnp.where(den > 0, 1.0 / den, 0.0)
    invb = jnp.repeat(inv, DH, axis=1)

    W03 = out_e_w[:64]
    W47 = out_e_w[64:]
    epre, stats = _epass1(edge_attr, we0, we1, W03, W47,
                          out_e_b.reshape(1, 128))
    mean = stats[0:1] / E
    var = stats[1:2] / E - mean * mean
    scale = g1e.reshape(1, 128) / jnp.sqrt(var + 1e-5)
    shift = b1e.reshape(1, 128) - mean * scale
    e = _epass2(epre, scale, shift)

    vebd = (jnp.transpose(VeRow, (1, 0, 2))[:, :, None, :]
            * jnp.eye(H, dtype=_f32)[:, None, :, None]).reshape(128, 128)
    dc0 = deg_coef[:, :, 0]
    dc1 = deg_coef[:, :, 1]
    h = _node(x, pv, pr, invb, deg.reshape(N, 1), vebd, out_h_w,
              out_h_b.reshape(1, 128), dc0, dc1, fc1w, fc1b.reshape(1, 256),
              fc2w, fc2b.reshape(1, 128), g1h.reshape(1, 128),
              b1h.reshape(1, 128), g2h.reshape(1, 128), b2h.reshape(1, 128))
    return (h, e)


# speedup vs baseline: 1.8262x; 1.8262x over previous
import functools

import jax
import jax.numpy as jnp
from jax import lax
from jax.experimental import pallas as pl
from jax.experimental.pallas import tpu as pltpu
from jax.experimental.pallas import tpu_sc as plsc

N = 10000
E = 320000
EH = E // 2
D = 128
H = 8
DH = 16
CLAMP = 5.0

G = 40
NTILES = 32
RPT = EH // NTILES
NCHUNK = RPT // G
ROWS_PT = 624
TAIL0 = 16 * ROWS_PT
TAILR = N - TAIL0

_f32 = jnp.float32


def _sc_body(src_hbm, dst_hbm, kvtab, qtab, epr, awt,
             we_out, pvpr_out, den_out,
             srcb0A, srcb1A, dstb0A, dstb1A,
             srcb0B, srcb1B, dstb0B, dstb1B,
             kvbuf0, kvbuf1, qbuf0, qbuf1, ebufA, ebufB, owbuf,
             pprb0, pprb1, denb0, denb1, awb,
             pvpr_sh, den_sh,
             sem_i, sem_g0, sem_g1, sem_e, sem_s0, sem_s1, sem_w):
    c = lax.axis_index("c")
    s = lax.axis_index("s")
    wid = c * 16 + s
    lanes = lax.iota(jnp.int32, 16)
    zero16 = jnp.zeros((16,), _f32)

    pltpu.sync_copy(awt, awb)

    def _zrow(i, _):
        for t in range(8):
            pprb0[i, pl.ds(t * 16, 16)] = zero16
            pprb1[i, pl.ds(t * 16, 16)] = zero16
        denb0[i, :] = zero16
        denb1[i, :] = zero16
        return 0
    lax.fori_loop(0, G, _zrow, 0)

    row0 = s * ROWS_PT
    for t in range(15):
        pltpu.sync_copy(pprb0, pvpr_sh.at[pl.ds(row0 + t * G, G)])
        pltpu.sync_copy(denb0, den_sh.at[pl.ds(row0 + t * G, G)])
    pltpu.sync_copy(pprb0, pvpr_sh.at[pl.ds(row0 + ROWS_PT - G, G)])
    pltpu.sync_copy(denb0, den_sh.at[pl.ds(row0 + ROWS_PT - G, G)])

    @pl.when(s == 0)
    def _zero_tail():
        pltpu.sync_copy(pprb0.at[pl.ds(0, TAILR)],
                        pvpr_sh.at[pl.ds(TAIL0, TAILR)])
        pltpu.sync_copy(denb0.at[pl.ds(0, TAILR)],
                        den_sh.at[pl.ds(TAIL0, TAILR)])

    plsc.subcore_barrier()

    rbase0 = wid * RPT
    _gdn = lax.GatherDimensionNumbers(
        offset_dims=(), collapsed_slice_dims=(0,), start_index_map=(0,))
    perms = [(lanes ^ sft)[:, None] for sft in (8, 4, 2, 1)]

    def _lanesum(t):
        for idxp in perms:
            t = t + lax.gather(t, idxp, _gdn, slice_sizes=(1,),
                               mode=lax.GatherScatterMode.PROMISE_IN_BOUNDS)
        return t

    himask = jnp.full((16,), -65536, jnp.int32)
    aws = [awb[j, :] for j in range(4)]
    hmasks = [jnp.where(lanes == j, 1.0, 0.0).astype(_f32) for j in range(4)]

    def _compute(kvbuf, qbuf, pprb, denb, ebuf, g):
        @plsc.parallel_loop(0, G, unroll=2)
        def edge_body(i):
            dvec = zero16
            for j in range(4):
                kv = kvbuf[i, pl.ds(j * 16, 16)]
                k = jax.lax.bitcast_convert_type(kv << 16, _f32)
                v = jax.lax.bitcast_convert_type(kv & himask, _f32)
                q = qbuf[i, pl.ds(j * 16, 16)]
                ee = ebuf[i, pl.ds(g * 64 + j * 16, 16)]
                ew_w = jax.lax.bitcast_convert_type(ee << 16, _f32)
                ew_b = jax.lax.bitcast_convert_type(ee & himask, _f32)
                ew = jnp.maximum((k + q) * ew_w + ew_b, 0.0)
                owbuf[i, pl.ds(g * 64 + j * 16, 16)] = ew
                sv = _lanesum(ew * aws[j])
                sv = jnp.minimum(jnp.maximum(sv, -CLAMP), CLAMP)
                pvec = jnp.exp(sv)
                pprb[i, pl.ds(j * 16, 16)] = v * pvec
                pprb[i, pl.ds(64 + j * 16, 16)] = ew * pvec
                dvec = dvec + pvec * hmasks[j]
            denb[i, :] = dvec

    IDX = {0: (srcb0A, srcb1A, dstb0A, dstb1A),
           1: (srcb0B, srcb1B, dstb0B, dstb1B)}
    EB = {0: ebufA, 1: ebufB}

    def fire_idx(tk, p):
        s0b, s1b, d0b, d1b = IDX[p]
        rb = rbase0 + tk * G
        pltpu.async_copy(src_hbm.at[pl.ds(rb, G)], s0b, sem_i)
        pltpu.async_copy(src_hbm.at[pl.ds(rb + EH, G)], s1b, sem_i)
        pltpu.async_copy(dst_hbm.at[pl.ds(rb, G)], d0b, sem_i)
        pltpu.async_copy(dst_hbm.at[pl.ds(rb + EH, G)], d1b, sem_i)

    def wait_idx(p):
        for b in IDX[p]:
            pltpu.make_async_copy(src_hbm.at[pl.ds(0, G)], b, sem_i).wait()

    def fire_epr(tk, p):
        pltpu.async_copy(epr.at[pl.ds(rbase0 + tk * G, G)], EB[p], sem_e)

    def wait_epr(p):
        pltpu.make_async_copy(epr.at[pl.ds(0, G)], EB[p], sem_e).wait()

    def fire_g(p):
        s0b, s1b, d0b, d1b = IDX[p]
        pltpu.async_copy(kvtab.at[s0b], kvbuf0, sem_g0)
        pltpu.async_copy(qtab.at[d0b], qbuf0, sem_g0)
        pltpu.async_copy(kvtab.at[s1b], kvbuf1, sem_g1)
        pltpu.async_copy(qtab.at[d1b], qbuf1, sem_g1)

    def wait_g(g):
        kvb, qb, sem = (kvbuf0, qbuf0, sem_g0) if g == 0 else \
                       (kvbuf1, qbuf1, sem_g1)
        pltpu.make_async_copy(kvtab.at[srcb0A], kvb, sem).wait()
        pltpu.make_async_copy(qtab.at[dstb0A], qb, sem).wait()

    def fire_s(g, p):
        d0b, d1b = IDX[p][2], IDX[p][3]
        db = d0b if g == 0 else d1b
        pprb, denb, sem = (pprb0, denb0, sem_s0) if g == 0 else \
                          (pprb1, denb1, sem_s1)
        pltpu.async_copy(pprb, pvpr_sh.at[db], sem, add=True)
        pltpu.async_copy(denb, den_sh.at[db], sem, add=True)

    def wait_s(g):
        pprb, denb, sem = (pprb0, denb0, sem_s0) if g == 0 else \
                          (pprb1, denb1, sem_s1)
        pltpu.make_async_copy(pprb, pvpr_sh.at[dstb0A], sem).wait()
        pltpu.make_async_copy(denb, den_sh.at[dstb0A], sem).wait()

    def fire_w(tk):
        pltpu.async_copy(owbuf, we_out.at[pl.ds(rbase0 + tk * G, G)], sem_w)

    def wait_w():
        pltpu.make_async_copy(owbuf, we_out.at[pl.ds(0, G)], sem_w).wait()

    def do_chunk(tk, cur, nxt, last):
        if not last:
            fire_idx(tk + 1, nxt)
        wait_w()
        wait_g(0)
        wait_epr(cur)
        wait_s(0)
        _compute(kvbuf0, qbuf0, pprb0, denb0, EB[cur], 0)
        fire_s(0, cur)
        if not last:
            wait_idx(nxt)
            fire_epr(tk + 1, nxt)
        wait_g(1)
        wait_s(1)
        _compute(kvbuf1, qbuf1, pprb1, denb1, EB[cur], 1)
        fire_s(1, cur)
        fire_w(tk)
        if not last:
            fire_g(nxt)

    fire_idx(0, 0)
    wait_idx(0)
    fire_s(0, 0)
    fire_s(1, 0)
    fire_w(0)
    fire_epr(0, 0)
    fire_g(0)

    def pair_body(m, _):
        do_chunk(2 * m, 0, 1, False)
        do_chunk(2 * m + 1, 1, 0, False)
        return 0

    lax.fori_loop(0, (NCHUNK - 1) // 2, pair_body, 0)
    do_chunk(NCHUNK - 1, 0, 1, True)

    wait_s(0)
    wait_s(1)
    wait_w()
    plsc.subcore_barrier()

    out_r0 = c * N + s * ROWS_PT
    pltpu.sync_copy(pvpr_sh.at[pl.ds(row0, ROWS_PT)],
                    pvpr_out.at[pl.ds(out_r0, ROWS_PT)])
    pltpu.sync_copy(den_sh.at[pl.ds(row0, ROWS_PT)],
                    den_out.at[pl.ds(out_r0, ROWS_PT)])

    @pl.when(s == 0)
    def _flush_tail():
        tr0 = c * N + TAIL0
        pltpu.sync_copy(pvpr_sh.at[pl.ds(TAIL0, TAILR)],
                        pvpr_out.at[pl.ds(tr0, TAILR)])
        pltpu.sync_copy(den_sh.at[pl.ds(TAIL0, TAILR)],
                        den_out.at[pl.ds(tr0, TAILR)])


_sc_edge = pl.kernel(
    _sc_body,
    out_type=[
        jax.ShapeDtypeStruct((EH, 128), _f32),
        jax.ShapeDtypeStruct((2 * N, 128), _f32),
        jax.ShapeDtypeStruct((2 * N, 16), _f32),
    ],
    mesh=plsc.VectorSubcoreMesh(core_axis_name="c", subcore_axis_name="s"),
    compiler_params=pltpu.CompilerParams(use_tc_tiling_on_sc=False),
    scratch_types=[
        pltpu.VMEM((G,), jnp.int32),
        pltpu.VMEM((G,), jnp.int32),
        pltpu.VMEM((G,), jnp.int32),
        pltpu.VMEM((G,), jnp.int32),
        pltpu.VMEM((G,), jnp.int32),
        pltpu.VMEM((G,), jnp.int32),
        pltpu.VMEM((G,), jnp.int32),
        pltpu.VMEM((G,), jnp.int32),
        pltpu.VMEM((G, 64), jnp.int32),
        pltpu.VMEM((G, 64), jnp.int32),
        pltpu.VMEM((G, 64), _f32),
        pltpu.VMEM((G, 64), _f32),
        pltpu.VMEM((G, 128), jnp.int32),
        pltpu.VMEM((G, 128), jnp.int32),
        pltpu.VMEM((G, 128), _f32),
        pltpu.VMEM((G, 128), _f32),
        pltpu.VMEM((G, 128), _f32),
        pltpu.VMEM((G, 16), _f32),
        pltpu.VMEM((G, 16), _f32),
        pltpu.VMEM((4, 16), _f32),
        pltpu.VMEM_SHARED((N, 128), _f32),
        pltpu.VMEM_SHARED((N, 16), _f32),
        pltpu.SemaphoreType.DMA,
        pltpu.SemaphoreType.DMA,
        pltpu.SemaphoreType.DMA,
        pltpu.SemaphoreType.DMA,
        pltpu.SemaphoreType.DMA,
        pltpu.SemaphoreType.DMA,
        pltpu.SemaphoreType.DMA,
    ],
)


_BN_ = 1000
_BE_ = 1000


def _pack_bf16(lo, hi):
    lb = jax.lax.bitcast_convert_type(lo, jnp.uint32)
    hb = jax.lax.bitcast_convert_type(hi, jnp.uint32)
    lr = (lb + jnp.uint32(0x7FFF) + ((lb >> 16) & jnp.uint32(1))) >> 16
    hr = (hb + jnp.uint32(0x7FFF) + ((hb >> 16) & jnp.uint32(1))) >> 16
    return jax.lax.bitcast_convert_type((hr << 16) | lr, jnp.int32)


def _proj_body(x, qw, kw, vw, qb, q0, q1, kv0, kv1):
    xb = x[...]
    qh = jnp.dot(xb, qw[...], preferred_element_type=_f32) + qb[...]
    kh = jnp.dot(xb, kw[...], preferred_element_type=_f32)
    vh = jnp.dot(xb, vw[...], preferred_element_type=_f32)
    q0[...] = qh[:, :64]
    q1[...] = qh[:, 64:]
    kvp = _pack_bf16(kh, vh)
    kv0[...] = kvp[:, :64]
    kv1[...] = kvp[:, 64:]


def _proj(x, Qw, Kw, Vw, Qb):
    nb = N // _BN_
    blk = lambda w: pl.BlockSpec((_BN_, w), lambda i: (i, 0))
    full = lambda a, b: pl.BlockSpec((a, b), lambda i: (0, 0))
    return pl.pallas_call(
        _proj_body,
        grid=(nb,),
        in_specs=[blk(128), full(128, 128), full(128, 128), full(128, 128),
                  full(1, 128)],
        out_specs=[blk(64)] * 4,
        out_shape=[jax.ShapeDtypeStruct((N, 64), _f32)] * 2
        + [jax.ShapeDtypeStruct((N, 64), jnp.int32)] * 2,
    )(x, Qw, Kw, Vw, Qb)


def _epr_body(ea_lo, ea_hi, ew, eb, o0, o1):
    m_lo = jnp.dot(ea_lo[...], ew[...], preferred_element_type=_f32) + eb[...]
    m_hi = jnp.dot(ea_hi[...], ew[...], preferred_element_type=_f32) + eb[...]
    o0[...] = jnp.concatenate(
        [_pack_bf16(m_lo[:, 0:64], m_lo[:, 64:128]),
         _pack_bf16(m_hi[:, 0:64], m_hi[:, 64:128])], axis=1)
    o1[...] = jnp.concatenate(
        [_pack_bf16(m_lo[:, 128:192], m_lo[:, 192:256]),
         _pack_bf16(m_hi[:, 128:192], m_hi[:, 192:256])], axis=1)


def _epr(edge_attr, Ewp, Ebp):
    nb = EH // _BE_
    return pl.pallas_call(
        _epr_body,
        grid=(nb,),
        in_specs=[pl.BlockSpec((_BE_, 128), lambda i: (i, 0)),
                  pl.BlockSpec((_BE_, 128), lambda i: (i + EH // _BE_, 0)),
                  pl.BlockSpec((128, 256), lambda i: (0, 0)),
                  pl.BlockSpec((1, 256), lambda i: (0, 0))],
        out_specs=[pl.BlockSpec((_BE_, 128), lambda i: (i, 0))] * 2,
        out_shape=[jax.ShapeDtypeStruct((EH, 128), jnp.int32)] * 2,
    )(edge_attr, edge_attr, Ewp, Ebp)


def _epass1_body(ea_lo, ea_hi, w0, w1, m0, m1, ob, eprep, stats):
    i = pl.program_id(0)
    w0b = w0[...]
    w1b = w1[...]
    m_lo = (ea_lo[...]
            + jnp.dot(w0b[:, :64], m0[...], preferred_element_type=_f32)
            + jnp.dot(w1b[:, :64], m1[...], preferred_element_type=_f32)
            + ob[...])
    m_hi = (ea_hi[...]
            + jnp.dot(w0b[:, 64:], m0[...], preferred_element_type=_f32)
            + jnp.dot(w1b[:, 64:], m1[...], preferred_element_type=_f32)
            + ob[...])
    eprep[...] = jnp.concatenate([m_lo, m_hi], axis=1).astype(jnp.bfloat16)

    @pl.when(i == 0)
    def _():
        stats[...] = jnp.zeros((8, 128), _f32)

    s0 = (jnp.sum(m_lo, axis=0, keepdims=True)
          + jnp.sum(m_hi, axis=0, keepdims=True))
    s1 = (jnp.sum(m_lo * m_lo, axis=0, keepdims=True)
          + jnp.sum(m_hi * m_hi, axis=0, keepdims=True))
    stats[0:1, :] += s0
    stats[1:2, :] += s1


def _epass1(edge_attr, we0, we1, W03, W47, oeb):
    nbh = EH // _BE_
    return pl.pallas_call(
        _epass1_body,
        grid=(nbh,),
        in_specs=[pl.BlockSpec((_BE_, 128), lambda i: (i, 0)),
                  pl.BlockSpec((_BE_, 128), lambda i: (i + EH // _BE_, 0)),
                  pl.BlockSpec((_BE_, 128), lambda i: (i, 0)),
                  pl.BlockSpec((_BE_, 128), lambda i: (i, 0)),
                  pl.BlockSpec((64, 128), lambda i: (0, 0)),
                  pl.BlockSpec((64, 128), lambda i: (0, 0)),
                  pl.BlockSpec((1, 128), lambda i: (0, 0))],
        out_specs=[pl.BlockSpec((_BE_, 256), lambda i: (i, 0)),
                   pl.BlockSpec((8, 128), lambda i: (0, 0))],
        out_shape=[jax.ShapeDtypeStruct((EH, 256), jnp.bfloat16),
                   jax.ShapeDtypeStruct((8, 128), _f32)],
    )(edge_attr, edge_attr, we0, we1, W03, W47, oeb)


def _epass2_body(epre, sc, sh, e):
    e[...] = epre[...].astype(_f32) * sc[...] + sh[...]


def _epass2(epre, scale, shift):
    nb = E // _BE_
    nbh = EH // _BE_
    return pl.pallas_call(
        _epass2_body,
        grid=(nb,),
        in_specs=[pl.BlockSpec((_BE_, 128), lambda i: (i % nbh, i // nbh)),
                  pl.BlockSpec((1, 128), lambda i: (0, 0)),
                  pl.BlockSpec((1, 128), lambda i: (0, 0))],
        out_specs=pl.BlockSpec((_BE_, 128), lambda i: (i, 0)),
        out_shape=jax.ShapeDtypeStruct((E, 128), _f32),
    )(epre, scale, shift)


def _node_body(x, pv, pr, invb, degc, vebd, ohw, ohb, dc0, dc1,
               f1w, f1b, f2w, f2b, g1, b1, g2, b2, out):
    xb = x[...]
    wv = pv[...] * invb[...] + jnp.dot(pr[...] * invb[...], vebd[...],
                                       preferred_element_type=_f32)
    ld = jnp.log(degc[...] + 1.0)
    hh = wv * dc0[...] + (wv * ld) * dc1[...]
    hh = jnp.dot(hh, ohw[...], preferred_element_type=_f32) + ohb[...]
    t = xb + hh
    mu = jnp.mean(t, axis=0, keepdims=True)
    var = jnp.mean(t * t, axis=0, keepdims=True) - mu * mu
    hn = g1[...] * (t - mu) / jnp.sqrt(var + 1e-5) + b1[...]
    u = jnp.maximum(jnp.dot(hn, f1w[...], preferred_element_type=_f32)
                    + f1b[...], 0.0)
    u = jnp.dot(u, f2w[...], preferred_element_type=_f32) + f2b[...] + hn
    mu2 = jnp.mean(u, axis=0, keepdims=True)
    var2 = jnp.mean(u * u, axis=0, keepdims=True) - mu2 * mu2
    out[...] = g2[...] * (u - mu2) / jnp.sqrt(var2 + 1e-5) + b2[...]


def _node(x, pv, pr, invb, degc, vebd, ohw, ohb, dc0, dc1,
          f1w, f1b, f2w, f2b, g1, b1, g2, b2):
    return pl.pallas_call(
        _node_body,
        out_shape=jax.ShapeDtypeStruct((N, 128), _f32),
    )(x, pv, pr, invb, degc, vebd, ohw, ohb, dc0, dc1,
      f1w, f1b, f2w, f2b, g1, b1, g2, b2)


def kernel(x, edge_attr, edge_index, deg, Qw, Qb, Kw, Ew, Eb, Vw, Aw, VeRow,
           out_h_w, out_h_b, out_e_w, out_e_b, deg_coef, g1h, b1h, g1e, b1e,
           fc1w, fc1b, fc2w, fc2b, g2h, b2h):
    src = edge_index[0]
    dst = edge_index[1]

    q0, q1, kv0, kv1 = _proj(x, Qw, Kw, Vw, Qb.reshape(1, 128))

    perm = jnp.array(
        [32 * h + t for h in range(4) for t in range(16)]
        + [32 * h + 16 + t for h in range(4) for t in range(16)]
        + [32 * h + t for h in range(4, 8) for t in range(16)]
        + [32 * h + 16 + t for h in range(4, 8) for t in range(16)],
        dtype=jnp.int32)
    Ewp = Ew[:, perm]
    Ebp = Eb[perm].reshape(1, 256)
    eprp0, eprp1 = _epr(edge_attr, Ewp, Ebp)

    awt = jnp.transpose(Aw[:, :, 0], (1, 0))

    we0, pvpr0, den0 = _sc_edge(src, dst, kv0, q0, eprp0, awt[0:4])
    we1, pvpr1, den1 = _sc_edge(src, dst, kv1, q1, eprp1, awt[4:8])

    a0 = pvpr0[:N] + pvpr0[N:]
    a1 = pvpr1[:N] + pvpr1[N:]
    pv = jnp.concatenate([a0[:, :64], a1[:, :64]], axis=1)
    pr = jnp.concatenate([a0[:, 64:], a1[:, 64:]], axis=1)
    den = jnp.concatenate([(den0[:N] + den0[N:])[:, :4],
                           (den1[:N] + den1[N:])[:, :4]], axis=1)
    inv = jnp.where(den > 0, 1.0 / den, 0.0)
    invb = jnp.repeat(inv, DH, axis=1)

    W03 = out_e_w[:64]
    W47 = out_e_w[64:]
    epre, stats = _epass1(edge_attr, we0, we1, W03, W47,
                          out_e_b.reshape(1, 128))
    mean = stats[0:1] / E
    var = stats[1:2] / E - mean * mean
    scale = g1e.reshape(1, 128) / jnp.sqrt(var + 1e-5)
    shift = b1e.reshape(1, 128) - mean * scale
    e = _epass2(epre, scale, shift)

    vebd = (jnp.transpose(VeRow, (1, 0, 2))[:, :, None, :]
            * jnp.eye(H, dtype=_f32)[:, None, :, None]).reshape(128, 128)
    dc0 = deg_coef[:, :, 0]
    dc1 = deg_coef[:, :, 1]
    h = _node(x, pv, pr, invb, deg.reshape(N, 1), vebd, out_h_w,
              out_h_b.reshape(1, 128), dc0, dc1, fc1w, fc1b.reshape(1, 256),
              fc2w, fc2b.reshape(1, 128), g1h.reshape(1, 128),
              b1h.reshape(1, 128), g2h.reshape(1, 128), b2h.reshape(1, 128))
    return (h, e)
